# baseline (speedup 1.0000x reference)
import functools

import jax
import jax.numpy as jnp
from jax import lax
from jax.experimental import pallas as pl
from jax.experimental.pallas import tpu as pltpu
from jax.experimental.pallas import tpu_sc as plsc

_B = 32
_K = 1024
_D = 32000
_KB = 128
_DB = 6400
_NUM_KB = _K // _KB
_NUM_DB = _D // _DB


def _dist_kernel(lat_ref, cb_ref, mind_ref, idx_ref, acc_ref, a2_ref):
    db = pl.program_id(0)
    kb = pl.program_id(1)
    lat = lat_ref[...]
    cb = cb_ref[...]
    dot = lax.dot_general(
        lat.astype(jnp.bfloat16), cb.astype(jnp.bfloat16),
        (((1,), (1,)), ((), ())),
        preferred_element_type=jnp.float32,
    )
    b2 = jnp.sum(cb * cb, axis=1)
    part = b2[None, :] - 2.0 * dot

    @pl.when(db == 0)
    def _():
        acc_ref[kb] = part

    @pl.when(db > 0)
    def _():
        acc_ref[kb] = acc_ref[kb] + part

    @pl.when(kb == 0)
    def _():
        a2p = jnp.sum(lat * lat, axis=1, keepdims=True)

        @pl.when(db == 0)
        def _():
            a2_ref[...] = a2p

        @pl.when(db > 0)
        def _():
            a2_ref[...] = a2_ref[...] + a2p

    @pl.when((db == _NUM_DB - 1) & (kb == _NUM_KB - 1))
    def _():
        runmin = None
        runarg = None
        for k2 in range(_NUM_KB):
            d2 = acc_ref[k2]
            bmin = jnp.min(d2, axis=1, keepdims=True)
            lane = lax.broadcasted_iota(jnp.int32, d2.shape, 1)
            barg = jnp.min(jnp.where(d2 == bmin, lane, _K), axis=1,
                           keepdims=True) + k2 * _KB
            if k2 == 0:
                runmin, runarg = bmin, barg
            else:
                better = bmin < runmin
                runarg = jnp.where(better, barg, runarg)
                runmin = jnp.where(better, bmin, runmin)
        idx_ref[...] = runarg
        mind_ref[...] = jnp.sqrt(jnp.maximum(a2_ref[...] + runmin, 0.0))


def _distance_argmin(lat_flat, cb_flat):
    return pl.pallas_call(
        _dist_kernel,
        grid=(_NUM_DB, _NUM_KB),
        in_specs=[
            pl.BlockSpec((_B, _DB), lambda db, kb: (0, db)),
            pl.BlockSpec((_KB, _DB), lambda db, kb: (kb, db)),
        ],
        out_specs=[
            pl.BlockSpec((_B, 1), lambda db, kb: (0, 0)),
            pl.BlockSpec((_B, 1), lambda db, kb: (0, 0)),
        ],
        out_shape=[
            jax.ShapeDtypeStruct((_B, 1), jnp.float32),
            jax.ShapeDtypeStruct((_B, 1), jnp.int32),
        ],
        scratch_shapes=[
            pltpu.VMEM((_NUM_KB, _B, _KB), jnp.float32),
            pltpu.VMEM((_B, 1), jnp.float32),
        ],
    )(lat_flat, cb_flat)


@functools.lru_cache(maxsize=None)
def _make_sc_gather():
    info = plsc.get_sparse_core_info()
    num_cores = info.num_cores

    @functools.partial(
        pl.kernel,
        mesh=plsc.VectorSubcoreMesh(core_axis_name="c", subcore_axis_name="s"),
        out_type=jax.ShapeDtypeStruct((_B, _D), jnp.float32),
        scratch_types=[
            pltpu.VMEM((1,), jnp.int32),
            pltpu.VMEM((1, _D), jnp.float32),
            pltpu.SemaphoreType.DMA,
        ],
    )
    def _sc_gather(table_hbm, idx_hbm, out_hbm, idx_v, rows_v, sem):
        wid = lax.axis_index("s") * num_cores + lax.axis_index("c")
        pltpu.sync_copy(idx_hbm.at[wid], idx_v)
        pltpu.async_copy(table_hbm.at[idx_v], rows_v, sem).wait()
        pltpu.sync_copy(rows_v, out_hbm.at[pl.ds(wid, 1)])

    return _sc_gather


def kernel(latent, codebook):
    B = latent.shape[0]
    K = codebook.shape[0]
    lat_flat = latent.reshape(B, -1).astype(jnp.float32)
    cb_flat = codebook.reshape(K, -1).astype(jnp.float32)

    mind, idx2 = _distance_argmin(lat_flat, cb_flat)
    idx = idx2.reshape(B)
    mind = mind.reshape(B)

    quant = _make_sc_gather()(cb_flat, idx2)
    quantized = quant.reshape(latent.shape).astype(latent.dtype)
    return (quantized, idx, mind)
